# baseline (speedup 1.0000x reference)
import jax
import jax.numpy as jnp
from jax import lax
from jax.experimental import pallas as pl
from jax.experimental.pallas import tpu as pltpu

P_MASK = 0.5
TIME_MASKING_PARA = 100
TIME_MASK_NUM = 1

NBUF = 4
SROWS = 3000


def _mask_bounds(B, T):
    key = jax.random.key(42)
    key, k_apply = jax.random.split(key)
    apply_mask = jax.random.uniform(k_apply) <= P_MASK
    starts_l, ends_l = [], []
    for _ in range(TIME_MASK_NUM):
        key, k_t, k_t0 = jax.random.split(key, 3)
        t = jax.random.randint(k_t, (), 0, TIME_MASKING_PARA + 1)
        t0s = jax.random.randint(k_t0, (B,), 0, T - TIME_MASKING_PARA)
        t_eff = jnp.where(apply_mask, t, 0)
        starts_l.append(t0s.astype(jnp.int32))
        ends_l.append((t0s + t_eff).astype(jnp.int32))
    return starts_l[0], ends_l[0]


def _make_body(B):
    def body(starts_ref, ends_ref, tt_ref, x_hbm, o_hbm, *rest):
        in_bufs = rest[0:NBUF]
        out_bufs = rest[NBUF:2 * NBUF]
        in_sems = rest[2 * NBUF:3 * NBUF]
        out_sems = rest[3 * NBUF:4 * NBUF]

        def in_copy(i, slot):
            return pltpu.make_async_copy(
                x_hbm.at[pl.ds(i * SROWS, SROWS), :], in_bufs[slot],
                in_sems[slot])

        def out_copy(i, slot):
            return pltpu.make_async_copy(
                out_bufs[slot], o_hbm.at[pl.ds(i * SROWS, SROWS), :],
                out_sems[slot])

        for i in range(NBUF):
            in_copy(i, i).start()
        for i in range(B):
            slot = i % NBUF
            in_copy(i, slot).wait()
            s = starts_ref[i]
            e = ends_ref[i]
            if i >= NBUF:
                out_copy(i - NBUF, slot).wait()
            tt = tt_ref[...]
            zero = (tt >= s) & (tt < e)
            out_bufs[slot][...] = jnp.where(zero, jnp.float32(0.0),
                                            in_bufs[slot][...])
            out_copy(i, slot).start()
            nxt = i + NBUF
            if nxt < B:
                in_copy(nxt, slot).start()
        for i in range(B - NBUF, B):
            out_copy(i, i % NBUF).wait()

    return body


def kernel(mel_batch):
    B, F, T = mel_batch.shape
    starts, ends = _mask_bounds(B, T)
    pos = lax.broadcasted_iota(jnp.int32, (SROWS, 128), 0) * 128 + \
        lax.broadcasted_iota(jnp.int32, (SROWS, 128), 1)
    tt = pos % T
    xf = mel_batch.reshape(B * F * T // 128, 128)
    out = pl.pallas_call(
        _make_body(B),
        grid=(),
        in_specs=[
            pl.BlockSpec(memory_space=pltpu.SMEM),
            pl.BlockSpec(memory_space=pltpu.SMEM),
            pl.BlockSpec(memory_space=pltpu.VMEM),
            pl.BlockSpec(memory_space=pl.ANY),
        ],
        out_specs=pl.BlockSpec(memory_space=pl.ANY),
        out_shape=jax.ShapeDtypeStruct((B * F * T // 128, 128), jnp.float32),
        scratch_shapes=(
            [pltpu.VMEM((SROWS, 128), jnp.float32) for _ in range(2 * NBUF)]
            + [pltpu.SemaphoreType.DMA for _ in range(2 * NBUF)]
        ),
    )(starts, ends, tt, xf)
    return out.reshape(B, F, T)


# speedup vs baseline: 3.8414x; 3.8414x over previous
import jax
import jax.numpy as jnp
from jax import lax
from jax.experimental import pallas as pl
from jax.experimental.pallas import tpu as pltpu

P_MASK = 0.5
TIME_MASKING_PARA = 100
TIME_MASK_NUM = 1

NBUF = 6


def _mask_bounds(B, T):
    key = jax.random.key(42)
    key, k_apply = jax.random.split(key)
    apply_mask = jax.random.uniform(k_apply) <= P_MASK
    starts_l, ends_l = [], []
    for _ in range(TIME_MASK_NUM):
        key, k_t, k_t0 = jax.random.split(key, 3)
        t = jax.random.randint(k_t, (), 0, TIME_MASKING_PARA + 1)
        t0s = jax.random.randint(k_t0, (B,), 0, T - TIME_MASKING_PARA)
        t_eff = jnp.where(apply_mask, t, 0)
        starts_l.append(t0s.astype(jnp.int32))
        ends_l.append((t0s + t_eff).astype(jnp.int32))
    return starts_l[0], ends_l[0]


def _make_body(B, F, T):
    def body(starts_ref, ends_ref, x_hbm, o_hbm, *rest):
        in_bufs = rest[0:NBUF]
        out_bufs = rest[NBUF:2 * NBUF]
        in_sems = rest[2 * NBUF:3 * NBUF]
        out_sems = rest[3 * NBUF:4 * NBUF]

        def in_copy(i, slot):
            return pltpu.make_async_copy(
                x_hbm.at[i], in_bufs[slot], in_sems[slot])

        def out_copy(i, slot):
            return pltpu.make_async_copy(
                out_bufs[slot], o_hbm.at[i], out_sems[slot])

        row = lax.broadcasted_iota(jnp.int32, (T, F), 0)
        for i in range(NBUF):
            in_copy(i, i).start(priority=i % 2)
        for i in range(B):
            slot = i % NBUF
            in_copy(i, slot).wait()
            s = starts_ref[i]
            e = ends_ref[i]
            if i >= NBUF:
                out_copy(i - NBUF, slot).wait()
            zero = (row >= s) & (row < e)
            out_bufs[slot][...] = jnp.where(zero, jnp.float32(0.0),
                                            in_bufs[slot][...])
            out_copy(i, slot).start(priority=slot % 2)
            nxt = i + NBUF
            if nxt < B:
                in_copy(nxt, slot).start(priority=slot % 2)
        for i in range(B - NBUF, B):
            out_copy(i, i % NBUF).wait()

    return body


def kernel(mel_batch):
    B, F, T = mel_batch.shape
    starts, ends = _mask_bounds(B, T)
    xt = jnp.transpose(mel_batch, (0, 2, 1))
    out_t = pl.pallas_call(
        _make_body(B, F, T),
        grid=(),
        in_specs=[
            pl.BlockSpec(memory_space=pltpu.SMEM),
            pl.BlockSpec(memory_space=pltpu.SMEM),
            pl.BlockSpec(memory_space=pl.ANY),
        ],
        out_specs=pl.BlockSpec(memory_space=pl.ANY),
        out_shape=jax.ShapeDtypeStruct((B, T, F), jnp.float32),
        scratch_shapes=(
            [pltpu.VMEM((T, F), jnp.float32) for _ in range(2 * NBUF)]
            + [pltpu.SemaphoreType.DMA for _ in range(2 * NBUF)]
        ),
    )(starts, ends, xt)
    return jnp.transpose(out_t, (0, 2, 1))


# speedup vs baseline: 3.8524x; 1.0029x over previous
import jax
import jax.numpy as jnp
from jax import lax
from jax.experimental import pallas as pl
from jax.experimental.pallas import tpu as pltpu

P_MASK = 0.5
TIME_MASKING_PARA = 100
TIME_MASK_NUM = 1

NBUF = 12
LAG = 4
WIN = 128


def _mask_bounds(B, T):
    key = jax.random.key(42)
    key, k_apply = jax.random.split(key)
    apply_mask = jax.random.uniform(k_apply) <= P_MASK
    starts_l, ends_l = [], []
    for _ in range(TIME_MASK_NUM):
        key, k_t, k_t0 = jax.random.split(key, 3)
        t = jax.random.randint(k_t, (), 0, TIME_MASKING_PARA + 1)
        t0s = jax.random.randint(k_t0, (B,), 0, T - TIME_MASKING_PARA)
        t_eff = jnp.where(apply_mask, t, 0)
        starts_l.append(t0s.astype(jnp.int32))
        ends_l.append((t0s + t_eff).astype(jnp.int32))
    return starts_l[0], ends_l[0]


def _make_body(B, F, T):
    def body(starts_ref, ends_ref, w0s_ref, x_hbm, o_hbm, *rest):
        bufs = rest[0:NBUF]
        in_sems = rest[NBUF:2 * NBUF]
        out_sems = rest[2 * NBUF:3 * NBUF]

        def in_copy(i, slot):
            return pltpu.make_async_copy(
                x_hbm.at[i], bufs[slot], in_sems[slot])

        def out_copy(i, slot):
            return pltpu.make_async_copy(
                bufs[slot], o_hbm.at[i], out_sems[slot])

        riota = lax.broadcasted_iota(jnp.int32, (WIN, F), 0)
        for i in range(NBUF):
            in_copy(i, i).start(priority=i % 2)
        for i in range(B):
            slot = i % NBUF
            in_copy(i, slot).wait()
            s = starts_ref[i]
            e = ends_ref[i]
            w0 = w0s_ref[i]
            rows = riota + w0
            slab = bufs[slot][pl.ds(w0, WIN), :]
            zero = (rows >= s) & (rows < e)
            bufs[slot][pl.ds(w0, WIN), :] = jnp.where(
                zero, jnp.float32(0.0), slab)
            out_copy(i, slot).start(priority=slot % 2)
            j = i - LAG
            if j >= 0:
                out_copy(j, j % NBUF).wait()
                nxt = j + NBUF
                if nxt < B:
                    in_copy(nxt, j % NBUF).start(priority=nxt % 2)
        for j in range(B - LAG, B):
            out_copy(j, j % NBUF).wait()

    return body


def kernel(mel_batch):
    B, F, T = mel_batch.shape
    starts, ends = _mask_bounds(B, T)
    w0s = jnp.minimum((starts // 8) * 8, T - WIN)
    xt = jnp.transpose(mel_batch, (0, 2, 1))
    out_t = pl.pallas_call(
        _make_body(B, F, T),
        grid=(),
        in_specs=[
            pl.BlockSpec(memory_space=pltpu.SMEM),
            pl.BlockSpec(memory_space=pltpu.SMEM),
            pl.BlockSpec(memory_space=pltpu.SMEM),
            pl.BlockSpec(memory_space=pl.ANY),
        ],
        out_specs=pl.BlockSpec(memory_space=pl.ANY),
        out_shape=jax.ShapeDtypeStruct((B, T, F), jnp.float32),
        scratch_shapes=(
            [pltpu.VMEM((T, F), jnp.float32) for _ in range(NBUF)]
            + [pltpu.SemaphoreType.DMA for _ in range(2 * NBUF)]
        ),
    )(starts, ends, w0s, xt)
    return jnp.transpose(out_t, (0, 2, 1))


# speedup vs baseline: 7.0338x; 1.8258x over previous
import jax
import jax.numpy as jnp
from jax import lax
from jax.experimental import pallas as pl
from jax.experimental.pallas import tpu as pltpu

P_MASK = 0.5
TIME_MASKING_PARA = 100
TIME_MASK_NUM = 1

NBUF = 12
LAG = 4
WIN = 128


def _mask_bounds(B, T):
    with jax.ensure_compile_time_eval():
        return _mask_bounds_traced(B, T)


def _mask_bounds_traced(B, T):
    key = jax.random.key(42)
    key, k_apply = jax.random.split(key)
    apply_mask = jax.random.uniform(k_apply) <= P_MASK
    starts_l, ends_l = [], []
    for _ in range(TIME_MASK_NUM):
        key, k_t, k_t0 = jax.random.split(key, 3)
        t = jax.random.randint(k_t, (), 0, TIME_MASKING_PARA + 1)
        t0s = jax.random.randint(k_t0, (B,), 0, T - TIME_MASKING_PARA)
        t_eff = jnp.where(apply_mask, t, 0)
        starts_l.append(t0s.astype(jnp.int32))
        ends_l.append((t0s + t_eff).astype(jnp.int32))
    return starts_l[0], ends_l[0]


def _make_body(B, F, T):
    def body(starts_ref, ends_ref, w0s_ref, x_hbm, o_hbm, *rest):
        bufs = rest[0:NBUF]
        in_sems = rest[NBUF:2 * NBUF]
        out_sems = rest[2 * NBUF:3 * NBUF]

        def in_copy(i, slot):
            return pltpu.make_async_copy(
                x_hbm.at[i], bufs[slot], in_sems[slot])

        def out_copy(i, slot):
            return pltpu.make_async_copy(
                bufs[slot], o_hbm.at[i], out_sems[slot])

        riota = lax.broadcasted_iota(jnp.int32, (WIN, F), 0)
        for i in range(NBUF):
            in_copy(i, i).start(priority=i % 2)
        for i in range(B):
            slot = i % NBUF
            in_copy(i, slot).wait()
            s = starts_ref[i]
            e = ends_ref[i]
            w0 = w0s_ref[i]
            rows = riota + w0
            slab = bufs[slot][pl.ds(w0, WIN), :]
            zero = (rows >= s) & (rows < e)
            bufs[slot][pl.ds(w0, WIN), :] = jnp.where(
                zero, jnp.float32(0.0), slab)
            out_copy(i, slot).start(priority=slot % 2)
            j = i - LAG
            if j >= 0:
                out_copy(j, j % NBUF).wait()
                nxt = j + NBUF
                if nxt < B:
                    in_copy(nxt, j % NBUF).start(priority=nxt % 2)
        for j in range(B - LAG, B):
            out_copy(j, j % NBUF).wait()

    return body


def kernel(mel_batch):
    B, F, T = mel_batch.shape
    starts, ends = _mask_bounds(B, T)
    w0s = jnp.minimum((starts // 8) * 8, T - WIN)
    xt = jnp.transpose(mel_batch, (0, 2, 1))
    out_t = pl.pallas_call(
        _make_body(B, F, T),
        grid=(),
        in_specs=[
            pl.BlockSpec(memory_space=pltpu.SMEM),
            pl.BlockSpec(memory_space=pltpu.SMEM),
            pl.BlockSpec(memory_space=pltpu.SMEM),
            pl.BlockSpec(memory_space=pl.ANY),
        ],
        out_specs=pl.BlockSpec(memory_space=pl.ANY),
        out_shape=jax.ShapeDtypeStruct((B, T, F), jnp.float32),
        scratch_shapes=(
            [pltpu.VMEM((T, F), jnp.float32) for _ in range(NBUF)]
            + [pltpu.SemaphoreType.DMA for _ in range(2 * NBUF)]
        ),
    )(starts, ends, w0s, xt)
    return jnp.transpose(out_t, (0, 2, 1))
